# Initial kernel scaffold; baseline (speedup 1.0000x reference)
import jax
import jax.numpy as jnp
from jax.experimental import pallas as pl


def kernel(x, wg, W1, b1, W2, b2):
    raise NotImplementedError("write your pallas kernel here")


# speedup vs baseline: 1.1785x; 1.1785x over previous
import functools

import jax
import jax.numpy as jnp
from jax import lax
from jax.experimental import pallas as pl
from jax.experimental.pallas import tpu as pltpu
from jax.experimental.pallas import tpu_sc as plsc

D = 768
F = 2048
E = 64
C = 80
T = 2048
NSLOT = E * C
NPAD = NSLOT + 8
NW = 32
TPW = T // NW


def _excl_cumsum(mask, tri):
    G = tri.shape[0]
    ng = T // G
    locs = []
    sums = []
    for g in range(ng):
        mg = mask[g * G:(g + 1) * G]
        locs.append(jnp.dot(tri, mg, preferred_element_type=jnp.float32))
        sums.append(jnp.sum(mg, axis=0, keepdims=True))
    off = jnp.zeros_like(sums[0])
    out = []
    for g in range(ng):
        out.append(locs[g] + off)
        off = off + sums[g]
    return jnp.concatenate(out, axis=0), off


def _gating_body(x_ref, wg_ref, s1d_ref, s2d_ref, s1c_ref, s2c_ref,
                 g1_ref, g2_ref, used_ref):
    xf = x_ref[...]
    logits = jnp.dot(xf, wg_ref[...], preferred_element_type=jnp.float32)
    m = jnp.max(logits, axis=1, keepdims=True)
    ex = jnp.exp(logits - m)
    gates = ex / jnp.sum(ex, axis=1, keepdims=True)

    iota_e = lax.broadcasted_iota(jnp.int32, (T, E), 1)
    m1 = jnp.max(gates, axis=1, keepdims=True)
    idx1 = jnp.min(jnp.where(gates == m1, iota_e, E), axis=1, keepdims=True)
    mask1 = (iota_e == idx1).astype(jnp.float32)
    gates2 = gates * (1.0 - mask1)
    m2 = jnp.max(gates2, axis=1, keepdims=True)
    idx2 = jnp.min(jnp.where(gates2 == m2, iota_e, E), axis=1, keepdims=True)
    mask2 = (iota_e == idx2).astype(jnp.float32)

    G = 256
    tri = (lax.broadcasted_iota(jnp.int32, (G, G), 0)
           > lax.broadcasted_iota(jnp.int32, (G, G), 1)).astype(jnp.float32)
    loc1, n1 = _excl_cumsum(mask1, tri)
    loc2, n2 = _excl_cumsum(mask2, tri)
    loc2 = loc2 + n1

    kept1 = mask1 * (loc1 < C).astype(jnp.float32)
    kept2 = mask2 * (loc2 < C).astype(jnp.float32)
    pos1 = jnp.sum(loc1 * kept1, axis=1, keepdims=True).astype(jnp.int32)
    pos2 = jnp.sum(loc2 * kept2, axis=1, keepdims=True).astype(jnp.int32)
    k1 = jnp.sum(kept1, axis=1, keepdims=True)
    k2 = jnp.sum(kept2, axis=1, keepdims=True)
    g1 = jnp.sum(gates * kept1, axis=1, keepdims=True)
    g2 = jnp.sum(gates * kept2, axis=1, keepdims=True)
    denom = g1 + g2 + 1e-9
    g1_ref[...] = g1 / denom
    g2_ref[...] = g2 / denom

    s1 = idx1 * C + pos1
    s2 = idx2 * C + pos2
    s1d_ref[...] = jnp.where(k1 > 0, s1, NSLOT)
    s2d_ref[...] = jnp.where(k2 > 0, s2, NSLOT)
    s1c_ref[...] = jnp.where(k1 > 0, s1, 0)
    s2c_ref[...] = jnp.where(k2 > 0, s2, 0)
    used_ref[...] = jnp.minimum(n1 + n2, float(C)).astype(jnp.int32)


def _gating_call(xf, wg):
    return pl.pallas_call(
        _gating_body,
        out_shape=(
            jax.ShapeDtypeStruct((T, 1), jnp.int32),
            jax.ShapeDtypeStruct((T, 1), jnp.int32),
            jax.ShapeDtypeStruct((T, 1), jnp.int32),
            jax.ShapeDtypeStruct((T, 1), jnp.int32),
            jax.ShapeDtypeStruct((T, 1), jnp.float32),
            jax.ShapeDtypeStruct((T, 1), jnp.float32),
            jax.ShapeDtypeStruct((1, E), jnp.int32),
        ),
    )(xf, wg)


def _dispatch_body(xf_hbm, s1_hbm, s2_hbm, out_hbm, rows_v, idx_v, sem):
    wid = lax.axis_index("s") * 2 + lax.axis_index("c")
    base = wid * TPW
    pltpu.sync_copy(xf_hbm.at[pl.ds(base, TPW)], rows_v)
    pltpu.sync_copy(s1_hbm.at[pl.ds(base, TPW)], idx_v)
    pltpu.async_copy(rows_v, out_hbm.at[idx_v], sem).wait()
    pltpu.sync_copy(s2_hbm.at[pl.ds(base, TPW)], idx_v)
    pltpu.async_copy(rows_v, out_hbm.at[idx_v], sem).wait()


def _dispatch_call(xf, s1d, s2d):
    mesh = plsc.VectorSubcoreMesh(core_axis_name="c", subcore_axis_name="s")
    f = functools.partial(
        pl.kernel,
        mesh=mesh,
        out_type=jax.ShapeDtypeStruct((NPAD, D), jnp.float32),
        scratch_types=[
            pltpu.VMEM((TPW, D), jnp.float32),
            pltpu.VMEM((TPW,), jnp.int32),
            pltpu.SemaphoreType.DMA,
        ],
    )(_dispatch_body)
    return f(xf, s1d, s2d)


def _ffn_body(used_ref, x_ref, w1_ref, b1_ref, w2_ref, b2_ref, o_ref):
    used = used_ref[0, 0, 0]
    rid = lax.broadcasted_iota(jnp.int32, (C, 1), 0)
    x = jnp.where(rid < used, x_ref[...], 0.0)
    h = jnp.dot(x, w1_ref[0], preferred_element_type=jnp.float32) + b1_ref[0]
    h = 0.5 * h * (1.0 + lax.erf(h * 0.7071067811865476))
    o_ref[...] = (jnp.dot(h, w2_ref[0], preferred_element_type=jnp.float32)
                  + b2_ref[0])


def _ffn_call(used, disp, W1, b1, W2, b2):
    return pl.pallas_call(
        _ffn_body,
        grid=(E,),
        in_specs=[
            pl.BlockSpec((1, 1, 1), lambda e: (e, 0, 0),
                         memory_space=pltpu.SMEM),
            pl.BlockSpec((C, D), lambda e: (e, 0)),
            pl.BlockSpec((1, D, F), lambda e: (e, 0, 0)),
            pl.BlockSpec((1, 1, F), lambda e: (e, 0, 0)),
            pl.BlockSpec((1, F, D), lambda e: (e, 0, 0)),
            pl.BlockSpec((1, 1, D), lambda e: (e, 0, 0)),
        ],
        out_specs=pl.BlockSpec((C, D), lambda e: (e, 0)),
        out_shape=jax.ShapeDtypeStruct((NSLOT, D), jnp.float32),
        compiler_params=pltpu.CompilerParams(
            dimension_semantics=("arbitrary",)),
    )(used, disp, W1, b1.reshape(E, 1, F), W2, b2.reshape(E, 1, D))


def _gather_body(eo_hbm, s1_hbm, s2_hbm, r1_hbm, r2_hbm, rows_v, idx_v, sem):
    wid = lax.axis_index("s") * 2 + lax.axis_index("c")
    base = wid * TPW
    pltpu.sync_copy(s1_hbm.at[pl.ds(base, TPW)], idx_v)
    pltpu.async_copy(eo_hbm.at[idx_v], rows_v, sem).wait()
    pltpu.sync_copy(rows_v, r1_hbm.at[pl.ds(base, TPW)])
    pltpu.sync_copy(s2_hbm.at[pl.ds(base, TPW)], idx_v)
    pltpu.async_copy(eo_hbm.at[idx_v], rows_v, sem).wait()
    pltpu.sync_copy(rows_v, r2_hbm.at[pl.ds(base, TPW)])


def _gather_call(eo, s1c, s2c):
    mesh = plsc.VectorSubcoreMesh(core_axis_name="c", subcore_axis_name="s")
    f = functools.partial(
        pl.kernel,
        mesh=mesh,
        out_type=(
            jax.ShapeDtypeStruct((T, D), jnp.float32),
            jax.ShapeDtypeStruct((T, D), jnp.float32),
        ),
        scratch_types=[
            pltpu.VMEM((TPW, D), jnp.float32),
            pltpu.VMEM((TPW,), jnp.int32),
            pltpu.SemaphoreType.DMA,
        ],
    )(_gather_body)
    return f(eo, s1c, s2c)


def _combine_body(r1_ref, r2_ref, g1_ref, g2_ref, o_ref):
    o_ref[...] = g1_ref[...] * r1_ref[...] + g2_ref[...] * r2_ref[...]


def _combine_call(r1, r2, g1n, g2n):
    return pl.pallas_call(
        _combine_body,
        out_shape=jax.ShapeDtypeStruct((T, D), jnp.float32),
    )(r1, r2, g1n, g2n)


def kernel(x, wg, W1, b1, W2, b2):
    xf = x.reshape(T, D)
    s1d, s2d, s1c, s2c, g1n, g2n, used = _gating_call(xf, wg)
    disp = _dispatch_call(xf, s1d.reshape(T), s2d.reshape(T))
    eo = _ffn_call(used.reshape(E, 1, 1), disp, W1, b1, W2, b2)
    r1, r2 = _gather_call(eo, s1c.reshape(T), s2c.reshape(T))
    out = _combine_call(r1, r2, g1n, g2n)
    return out.reshape(1, T, D)


# speedup vs baseline: 1.1931x; 1.0124x over previous
import functools

import jax
import jax.numpy as jnp
from jax import lax
from jax.experimental import pallas as pl
from jax.experimental.pallas import tpu as pltpu
from jax.experimental.pallas import tpu_sc as plsc

D = 768
F = 2048
E = 64
C = 80
T = 2048
NSLOT = E * C
NPAD = NSLOT + 8
NW = 32
TPW = T // NW


def _excl_cumsum(mask, tri):
    G = tri.shape[0]
    ng = T // G
    locs = []
    sums = []
    for g in range(ng):
        mg = mask[g * G:(g + 1) * G]
        locs.append(jnp.dot(tri, mg, preferred_element_type=jnp.float32))
        sums.append(jnp.sum(mg, axis=0, keepdims=True))
    off = jnp.zeros_like(sums[0])
    out = []
    for g in range(ng):
        out.append(locs[g] + off)
        off = off + sums[g]
    return jnp.concatenate(out, axis=0), off


def _gating_body(x_ref, wg_ref, s1d_ref, s2d_ref, s1c_ref, s2c_ref,
                 g1_ref, g2_ref, used_ref):
    xf = x_ref[...]
    logits = jnp.dot(xf, wg_ref[...], preferred_element_type=jnp.float32)
    m = jnp.max(logits, axis=1, keepdims=True)
    ex = jnp.exp(logits - m)
    gates = ex / jnp.sum(ex, axis=1, keepdims=True)

    iota_e = lax.broadcasted_iota(jnp.int32, (T, E), 1)
    m1 = jnp.max(gates, axis=1, keepdims=True)
    idx1 = jnp.min(jnp.where(gates == m1, iota_e, E), axis=1, keepdims=True)
    mask1 = (iota_e == idx1).astype(jnp.float32)
    gates2 = gates * (1.0 - mask1)
    m2 = jnp.max(gates2, axis=1, keepdims=True)
    idx2 = jnp.min(jnp.where(gates2 == m2, iota_e, E), axis=1, keepdims=True)
    mask2 = (iota_e == idx2).astype(jnp.float32)

    G = 256
    tri = (lax.broadcasted_iota(jnp.int32, (G, G), 0)
           > lax.broadcasted_iota(jnp.int32, (G, G), 1)).astype(jnp.float32)
    loc1, n1 = _excl_cumsum(mask1, tri)
    loc2, n2 = _excl_cumsum(mask2, tri)
    loc2 = loc2 + n1

    kept1 = mask1 * (loc1 < C).astype(jnp.float32)
    kept2 = mask2 * (loc2 < C).astype(jnp.float32)
    pos1 = jnp.sum(loc1 * kept1, axis=1, keepdims=True).astype(jnp.int32)
    pos2 = jnp.sum(loc2 * kept2, axis=1, keepdims=True).astype(jnp.int32)
    k1 = jnp.sum(kept1, axis=1, keepdims=True)
    k2 = jnp.sum(kept2, axis=1, keepdims=True)
    g1 = jnp.sum(gates * kept1, axis=1, keepdims=True)
    g2 = jnp.sum(gates * kept2, axis=1, keepdims=True)
    denom = g1 + g2 + 1e-9
    ones16 = jnp.ones((1, 16), jnp.float32)
    g1_ref[...] = (g1 / denom) * ones16
    g2_ref[...] = (g2 / denom) * ones16

    s1 = idx1 * C + pos1
    s2 = idx2 * C + pos2
    s1d_ref[...] = jnp.where(k1 > 0, s1, NSLOT)
    s2d_ref[...] = jnp.where(k2 > 0, s2, NSLOT)
    s1c_ref[...] = jnp.where(k1 > 0, s1, 0)
    s2c_ref[...] = jnp.where(k2 > 0, s2, 0)
    used_ref[...] = jnp.minimum(n1 + n2, float(C)).astype(jnp.int32)


def _gating_call(xf, wg):
    return pl.pallas_call(
        _gating_body,
        out_shape=(
            jax.ShapeDtypeStruct((T, 1), jnp.int32),
            jax.ShapeDtypeStruct((T, 1), jnp.int32),
            jax.ShapeDtypeStruct((T, 1), jnp.int32),
            jax.ShapeDtypeStruct((T, 1), jnp.int32),
            jax.ShapeDtypeStruct((T, 16), jnp.float32),
            jax.ShapeDtypeStruct((T, 16), jnp.float32),
            jax.ShapeDtypeStruct((1, E), jnp.int32),
        ),
    )(xf, wg)


def _dispatch_body(xf_hbm, s1_hbm, s2_hbm, out_hbm, rows_v, idx_v, sem):
    wid = lax.axis_index("s") * 2 + lax.axis_index("c")
    base = wid * TPW
    pltpu.sync_copy(xf_hbm.at[pl.ds(base, TPW)], rows_v)
    pltpu.sync_copy(s1_hbm.at[pl.ds(base, TPW)], idx_v)
    pltpu.async_copy(rows_v, out_hbm.at[idx_v], sem).wait()
    pltpu.sync_copy(s2_hbm.at[pl.ds(base, TPW)], idx_v)
    pltpu.async_copy(rows_v, out_hbm.at[idx_v], sem).wait()


def _dispatch_call(xf, s1d, s2d):
    mesh = plsc.VectorSubcoreMesh(core_axis_name="c", subcore_axis_name="s")
    f = functools.partial(
        pl.kernel,
        mesh=mesh,
        out_type=jax.ShapeDtypeStruct((NPAD, D), jnp.float32),
        scratch_types=[
            pltpu.VMEM((TPW, D), jnp.float32),
            pltpu.VMEM((TPW,), jnp.int32),
            pltpu.SemaphoreType.DMA,
        ],
    )(_dispatch_body)
    return f(xf, s1d, s2d)


def _ffn_body(used_ref, x_ref, w1_ref, b1_ref, w2_ref, b2_ref, o_ref):
    used = used_ref[0, 0, 0]
    rid = lax.broadcasted_iota(jnp.int32, (C, 1), 0)
    x = jnp.where(rid < used, x_ref[...], 0.0)
    h = jnp.dot(x, w1_ref[0], preferred_element_type=jnp.float32) + b1_ref[0]
    h = 0.5 * h * (1.0 + lax.erf(h * 0.7071067811865476))
    o_ref[...] = (jnp.dot(h, w2_ref[0], preferred_element_type=jnp.float32)
                  + b2_ref[0])


def _ffn_call(used, disp, W1, b1, W2, b2):
    return pl.pallas_call(
        _ffn_body,
        grid=(E,),
        in_specs=[
            pl.BlockSpec((1, 1, 1), lambda e: (e, 0, 0),
                         memory_space=pltpu.SMEM),
            pl.BlockSpec((C, D), lambda e: (e, 0)),
            pl.BlockSpec((1, D, F), lambda e: (e, 0, 0)),
            pl.BlockSpec((1, 1, F), lambda e: (e, 0, 0)),
            pl.BlockSpec((1, F, D), lambda e: (e, 0, 0)),
            pl.BlockSpec((1, 1, D), lambda e: (e, 0, 0)),
        ],
        out_specs=pl.BlockSpec((C, D), lambda e: (e, 0)),
        out_shape=jax.ShapeDtypeStruct((NSLOT, D), jnp.float32),
        compiler_params=pltpu.CompilerParams(
            dimension_semantics=("arbitrary",)),
    )(used, disp, W1, b1.reshape(E, 1, F), W2, b2.reshape(E, 1, D))


def _gather_body(eo_hbm, s1_hbm, s2_hbm, g1_hbm, g2_hbm, out_hbm,
                 r1_v, r2_v, idx_v, g1_v, g2_v, sem):
    wid = lax.axis_index("s") * 2 + lax.axis_index("c")
    base = wid * TPW
    pltpu.sync_copy(s1_hbm.at[pl.ds(base, TPW)], idx_v)
    pltpu.async_copy(eo_hbm.at[idx_v], r1_v, sem).wait()
    pltpu.sync_copy(s2_hbm.at[pl.ds(base, TPW)], idx_v)
    pltpu.async_copy(eo_hbm.at[idx_v], r2_v, sem).wait()
    pltpu.sync_copy(g1_hbm.at[pl.ds(base, TPW)], g1_v)
    pltpu.sync_copy(g2_hbm.at[pl.ds(base, TPW)], g2_v)

    def tok(t, _):
        g1s = g1_v[t, pl.ds(0, 16)]
        g2s = g2_v[t, pl.ds(0, 16)]
        for j in range(D // 16):
            a = r1_v[t, pl.ds(j * 16, 16)]
            b = r2_v[t, pl.ds(j * 16, 16)]
            r1_v[t, pl.ds(j * 16, 16)] = g1s * a + g2s * b
        return 0

    lax.fori_loop(0, TPW, tok, 0)
    pltpu.sync_copy(r1_v, out_hbm.at[pl.ds(base, TPW)])


def _gather_call(eo, s1c, s2c, g1n, g2n):
    mesh = plsc.VectorSubcoreMesh(core_axis_name="c", subcore_axis_name="s")
    f = functools.partial(
        pl.kernel,
        mesh=mesh,
        out_type=jax.ShapeDtypeStruct((T, D), jnp.float32),
        scratch_types=[
            pltpu.VMEM((TPW, D), jnp.float32),
            pltpu.VMEM((TPW, D), jnp.float32),
            pltpu.VMEM((TPW,), jnp.int32),
            pltpu.VMEM((TPW, 16), jnp.float32),
            pltpu.VMEM((TPW, 16), jnp.float32),
            pltpu.SemaphoreType.DMA,
        ],
    )(_gather_body)
    return f(eo, s1c, s2c, g1n, g2n)


def kernel(x, wg, W1, b1, W2, b2):
    xf = x.reshape(T, D)
    s1d, s2d, s1c, s2c, g1n, g2n, used = _gating_call(xf, wg)
    disp = _dispatch_call(xf, s1d.reshape(T), s2d.reshape(T))
    eo = _ffn_call(used.reshape(E, 1, 1), disp, W1, b1, W2, b2)
    out = _gather_call(eo, s1c.reshape(T), s2c.reshape(T), g1n, g2n)
    return out.reshape(1, T, D)


# speedup vs baseline: 1.2022x; 1.0077x over previous
import functools

import jax
import jax.numpy as jnp
from jax import lax
from jax.experimental import pallas as pl
from jax.experimental.pallas import tpu as pltpu
from jax.experimental.pallas import tpu_sc as plsc

D = 768
F = 2048
E = 64
C = 80
T = 2048
NSLOT = E * C
NPAD = NSLOT + 8
NW = 32
TPW = T // NW


def _excl_cumsum(mask, tri):
    G = tri.shape[0]
    ng = T // G
    locs = []
    sums = []
    for g in range(ng):
        mg = mask[g * G:(g + 1) * G]
        locs.append(jnp.dot(tri, mg, preferred_element_type=jnp.float32))
        sums.append(jnp.sum(mg, axis=0, keepdims=True))
    off = jnp.zeros_like(sums[0])
    out = []
    for g in range(ng):
        out.append(locs[g] + off)
        off = off + sums[g]
    return jnp.concatenate(out, axis=0), off


def _gating_body(x_ref, wg_ref, s1d_ref, s2d_ref, s1c_ref, s2c_ref,
                 g1_ref, g2_ref, used_ref):
    xf = x_ref[...]
    logits = jnp.dot(xf, wg_ref[...], preferred_element_type=jnp.float32)
    m = jnp.max(logits, axis=1, keepdims=True)
    ex = jnp.exp(logits - m)
    gates = ex / jnp.sum(ex, axis=1, keepdims=True)

    iota_e = lax.broadcasted_iota(jnp.int32, (T, E), 1)
    m1 = jnp.max(gates, axis=1, keepdims=True)
    idx1 = jnp.min(jnp.where(gates == m1, iota_e, E), axis=1, keepdims=True)
    mask1 = (iota_e == idx1).astype(jnp.float32)
    gates2 = gates * (1.0 - mask1)
    m2 = jnp.max(gates2, axis=1, keepdims=True)
    idx2 = jnp.min(jnp.where(gates2 == m2, iota_e, E), axis=1, keepdims=True)
    mask2 = (iota_e == idx2).astype(jnp.float32)

    G = 256
    tri = (lax.broadcasted_iota(jnp.int32, (G, G), 0)
           > lax.broadcasted_iota(jnp.int32, (G, G), 1)).astype(jnp.float32)
    loc1, n1 = _excl_cumsum(mask1, tri)
    loc2, n2 = _excl_cumsum(mask2, tri)
    loc2 = loc2 + n1

    kept1 = mask1 * (loc1 < C).astype(jnp.float32)
    kept2 = mask2 * (loc2 < C).astype(jnp.float32)
    pos1 = jnp.sum(loc1 * kept1, axis=1, keepdims=True).astype(jnp.int32)
    pos2 = jnp.sum(loc2 * kept2, axis=1, keepdims=True).astype(jnp.int32)
    k1 = jnp.sum(kept1, axis=1, keepdims=True)
    k2 = jnp.sum(kept2, axis=1, keepdims=True)
    g1 = jnp.sum(gates * kept1, axis=1, keepdims=True)
    g2 = jnp.sum(gates * kept2, axis=1, keepdims=True)
    denom = g1 + g2 + 1e-9
    ones16 = jnp.ones((1, 16), jnp.float32)
    g1_ref[...] = (g1 / denom) * ones16
    g2_ref[...] = (g2 / denom) * ones16

    s1 = idx1 * C + pos1
    s2 = idx2 * C + pos2
    s1d_ref[...] = jnp.where(k1 > 0, s1, NSLOT)
    s2d_ref[...] = jnp.where(k2 > 0, s2, NSLOT)
    s1c_ref[...] = jnp.where(k1 > 0, s1, 0)
    s2c_ref[...] = jnp.where(k2 > 0, s2, 0)
    used_ref[...] = jnp.minimum(n1 + n2, float(C)).astype(jnp.int32)


def _gating_call(xf, wg):
    return pl.pallas_call(
        _gating_body,
        out_shape=(
            jax.ShapeDtypeStruct((T, 1), jnp.int32),
            jax.ShapeDtypeStruct((T, 1), jnp.int32),
            jax.ShapeDtypeStruct((T, 1), jnp.int32),
            jax.ShapeDtypeStruct((T, 1), jnp.int32),
            jax.ShapeDtypeStruct((T, 16), jnp.float32),
            jax.ShapeDtypeStruct((T, 16), jnp.float32),
            jax.ShapeDtypeStruct((1, E), jnp.int32),
        ),
    )(xf, wg)


def _dispatch_body(xf_hbm, s1_hbm, s2_hbm, out_hbm, rows_v, idx1_v, idx2_v,
                   sem1, sem2):
    wid = lax.axis_index("s") * 2 + lax.axis_index("c")
    base = wid * TPW
    pltpu.sync_copy(s1_hbm.at[pl.ds(base, TPW)], idx1_v)
    pltpu.sync_copy(s2_hbm.at[pl.ds(base, TPW)], idx2_v)
    pltpu.sync_copy(xf_hbm.at[pl.ds(base, TPW)], rows_v)
    c1 = pltpu.async_copy(rows_v, out_hbm.at[idx1_v], sem1)
    c2 = pltpu.async_copy(rows_v, out_hbm.at[idx2_v], sem2)
    c1.wait()
    c2.wait()


def _dispatch_call(xf, s1d, s2d):
    mesh = plsc.VectorSubcoreMesh(core_axis_name="c", subcore_axis_name="s")
    f = functools.partial(
        pl.kernel,
        mesh=mesh,
        out_type=jax.ShapeDtypeStruct((NPAD, D), jnp.float32),
        scratch_types=[
            pltpu.VMEM((TPW, D), jnp.float32),
            pltpu.VMEM((TPW,), jnp.int32),
            pltpu.VMEM((TPW,), jnp.int32),
            pltpu.SemaphoreType.DMA,
            pltpu.SemaphoreType.DMA,
        ],
    )(_dispatch_body)
    return f(xf, s1d, s2d)


def _ffn_body(used_ref, x_ref, w1_ref, b1_ref, w2_ref, b2_ref, o_ref):
    used = used_ref[0, 0, 0]
    rid = lax.broadcasted_iota(jnp.int32, (C, 1), 0)
    x = jnp.where(rid < used, x_ref[...], 0.0)
    h = jnp.dot(x, w1_ref[0], preferred_element_type=jnp.float32) + b1_ref[0]
    h = 0.5 * h * (1.0 + lax.erf(h * 0.7071067811865476))
    o_ref[...] = (jnp.dot(h, w2_ref[0], preferred_element_type=jnp.float32)
                  + b2_ref[0])


def _ffn_call(used, disp, W1, b1, W2, b2):
    return pl.pallas_call(
        _ffn_body,
        grid=(E,),
        in_specs=[
            pl.BlockSpec((1, 1, 1), lambda e: (e, 0, 0),
                         memory_space=pltpu.SMEM),
            pl.BlockSpec((C, D), lambda e: (e, 0)),
            pl.BlockSpec((1, D, F), lambda e: (e, 0, 0)),
            pl.BlockSpec((1, 1, F), lambda e: (e, 0, 0)),
            pl.BlockSpec((1, F, D), lambda e: (e, 0, 0)),
            pl.BlockSpec((1, 1, D), lambda e: (e, 0, 0)),
        ],
        out_specs=pl.BlockSpec((C, D), lambda e: (e, 0)),
        out_shape=jax.ShapeDtypeStruct((NSLOT, D), jnp.float32),
        compiler_params=pltpu.CompilerParams(
            dimension_semantics=("arbitrary",)),
    )(used, disp, W1, b1.reshape(E, 1, F), W2, b2.reshape(E, 1, D))


def _gather_body(eo_hbm, s1_hbm, s2_hbm, g1_hbm, g2_hbm, out_hbm,
                 r1_v, r2_v, idx1_v, idx2_v, g1_v, g2_v, sem1, sem2):
    wid = lax.axis_index("s") * 2 + lax.axis_index("c")
    base = wid * TPW
    pltpu.sync_copy(s1_hbm.at[pl.ds(base, TPW)], idx1_v)
    pltpu.sync_copy(s2_hbm.at[pl.ds(base, TPW)], idx2_v)
    c1 = pltpu.async_copy(eo_hbm.at[idx1_v], r1_v, sem1)
    c2 = pltpu.async_copy(eo_hbm.at[idx2_v], r2_v, sem2)
    pltpu.sync_copy(g1_hbm.at[pl.ds(base, TPW)], g1_v)
    pltpu.sync_copy(g2_hbm.at[pl.ds(base, TPW)], g2_v)
    c1.wait()
    c2.wait()

    def tok(t, _):
        g1s = g1_v[t, pl.ds(0, 16)]
        g2s = g2_v[t, pl.ds(0, 16)]
        for j in range(D // 16):
            a = r1_v[t, pl.ds(j * 16, 16)]
            b = r2_v[t, pl.ds(j * 16, 16)]
            r1_v[t, pl.ds(j * 16, 16)] = g1s * a + g2s * b
        return 0

    lax.fori_loop(0, TPW, tok, 0)
    pltpu.sync_copy(r1_v, out_hbm.at[pl.ds(base, TPW)])


def _gather_call(eo, s1c, s2c, g1n, g2n):
    mesh = plsc.VectorSubcoreMesh(core_axis_name="c", subcore_axis_name="s")
    f = functools.partial(
        pl.kernel,
        mesh=mesh,
        out_type=jax.ShapeDtypeStruct((T, D), jnp.float32),
        scratch_types=[
            pltpu.VMEM((TPW, D), jnp.float32),
            pltpu.VMEM((TPW, D), jnp.float32),
            pltpu.VMEM((TPW,), jnp.int32),
            pltpu.VMEM((TPW,), jnp.int32),
            pltpu.VMEM((TPW, 16), jnp.float32),
            pltpu.VMEM((TPW, 16), jnp.float32),
            pltpu.SemaphoreType.DMA,
            pltpu.SemaphoreType.DMA,
        ],
    )(_gather_body)
    return f(eo, s1c, s2c, g1n, g2n)


def kernel(x, wg, W1, b1, W2, b2):
    xf = x.reshape(T, D)
    s1d, s2d, s1c, s2c, g1n, g2n, used = _gating_call(xf, wg)
    disp = _dispatch_call(xf, s1d.reshape(T), s2d.reshape(T))
    eo = _ffn_call(used.reshape(E, 1, 1), disp, W1, b1, W2, b2)
    out = _gather_call(eo, s1c.reshape(T), s2c.reshape(T), g1n, g2n)
    return out.reshape(1, T, D)
